# Initial kernel scaffold; baseline (speedup 1.0000x reference)
import jax
import jax.numpy as jnp
from jax.experimental import pallas as pl


def kernel(coords, feats, q_w, q_b, k_w, k_b, v_w, v_b, lin1_w, lin1_b, lin2_w, lin2_b, we1_w, we1_b, we2_w, we2_b, ln_att_g, ln_att_b, pe1_w, pe1_b, pe2_w, pe2_b, sg1_w, sg1_b, sg2_w, sg2_b, ln_ff_g, ln_ff_b):
    raise NotImplementedError("write your pallas kernel here")


# speedup vs baseline: 6.6541x; 6.6541x over previous
import functools

import jax
import jax.numpy as jnp
from jax import lax
from jax.experimental import pallas as pl
from jax.experimental.pallas import tpu as pltpu
from jax.experimental.pallas import tpu_sc as plsc

N = 10000
NP = 10240
D = 128
K = 32
T = 34
QB = 128
BN = 64
RB = 1024
NC, NS = 2, 16
NW = NC * NS
NPW = NP // NW
GCH = 16

_BIGF = 1e9
_INF = 3e38


def _knn_body(coords4_ref, ct_ref, lin1w4_ref, idx_ref, p_ref, d_scr):
    i = pl.program_id(0)
    q = coords4_ref[pl.ds(i * QB, QB), :]
    ct = ct_ref[...]
    qc = jnp.dot(q, ct, preferred_element_type=jnp.float32)
    qsq = jnp.sum(q * q, axis=1, keepdims=True)
    csq = jnp.sum(ct * ct, axis=0, keepdims=True)
    d_scr[...] = qsq - 2.0 * qc + csq

    col64 = lax.broadcasted_iota(jnp.int32, (QB, 64), 1)

    def step(t, e):
        d = d_scr[...]
        m = jnp.min(d, axis=1, keepdims=True)
        iota_j = lax.broadcasted_iota(jnp.int32, (QB, NP), 1)
        is_min = d == m
        jsel = jnp.min(jnp.where(is_min, iota_j, jnp.int32(2**30)), axis=1,
                       keepdims=True)
        d_scr[...] = jnp.where(is_min & (iota_j == jsel), _INF, d)
        return jnp.where(col64 == t, jsel, e)

    e = lax.fori_loop(0, T, step, jnp.zeros((QB, 64), jnp.int32))

    rowid = i * QB + lax.broadcasted_iota(jnp.int32, (QB, 1), 0)
    self_hit = (e == rowid) & (col64 < T)
    p = jnp.min(jnp.where(self_hit, col64, jnp.int32(10 * T)), axis=1,
                keepdims=True)
    idx = e[:, 0:K]
    idx_g = jnp.where(col64[:, 0:K] < p, idx, e[:, 1:K + 1])
    idx_ref[:, 0:K] = idx
    idx_ref[:, K:2 * K] = idx_g
    p_ref[...] = jnp.dot(q, lin1w4_ref[...],
                         preferred_element_type=jnp.float32)


def _knn(coords4, lin1_w4):
    ct = coords4[:, 0:4].T
    return pl.pallas_call(
        _knn_body,
        grid=(NP // QB,),
        in_specs=[
            pl.BlockSpec(memory_space=pltpu.VMEM),
            pl.BlockSpec(memory_space=pltpu.VMEM),
            pl.BlockSpec(memory_space=pltpu.VMEM),
        ],
        out_specs=[
            pl.BlockSpec((QB, 128), lambda i: (i, 0)),
            pl.BlockSpec((QB, D), lambda i: (i, 0)),
        ],
        out_shape=[
            jax.ShapeDtypeStruct((NP, 128), jnp.int32),
            jax.ShapeDtypeStruct((NP, D), jnp.float32),
        ],
        scratch_shapes=[pltpu.VMEM((QB, NP), jnp.float32)],
    )(coords4, ct, lin1_w4)


def _gather2_body(t1_hbm, t2_hbm, idx_hbm, o1_hbm, o2_hbm, idxv, buf, sem):
    wid = lax.axis_index("s") * NC + lax.axis_index("c")
    rows_per_w = NPW * K
    base = wid * rows_per_w
    ch = GCH * K

    @pl.loop(0, rows_per_w, step=ch)
    def _chunk(off):
        start = base + off
        pltpu.sync_copy(idx_hbm.at[pl.ds(start, ch)], idxv)
        pltpu.async_copy(t1_hbm.at[idxv], buf, sem).wait()
        pltpu.sync_copy(buf, o1_hbm.at[pl.ds(start, ch)])
        pltpu.async_copy(t2_hbm.at[idxv], buf, sem).wait()
        pltpu.sync_copy(buf, o2_hbm.at[pl.ds(start, ch)])


def _gather2(tbl1, tbl2, idx_flat):
    mesh = plsc.VectorSubcoreMesh(core_axis_name="c", subcore_axis_name="s")
    fn = pl.kernel(
        _gather2_body,
        out_type=[
            jax.ShapeDtypeStruct((NP * K, D), jnp.float32),
            jax.ShapeDtypeStruct((NP * K, D), jnp.float32),
        ],
        mesh=mesh,
        scratch_types=[
            pltpu.VMEM((GCH * K,), jnp.int32),
            pltpu.VMEM((GCH * K, D), jnp.float32),
            pltpu.SemaphoreType.DMA,
        ],
    )
    return fn(tbl1, tbl2, idx_flat)


def _ln(x, g, b):
    m = jnp.mean(x, -1, keepdims=True)
    v = jnp.mean((x - m) ** 2, -1, keepdims=True)
    return (x - m) / jnp.sqrt(v + 1e-5) * g + b


def _attn_body(nf_ref, pnc_ref, feats_ref, pself_ref, coords4_ref,
               lin1b_ref, lin2w_ref, lin2b_ref,
               qw_ref, qb_ref, kw_ref, kb_ref, vw_ref, vb_ref,
               we1w_ref, we1b_ref, we2w_ref, we2b_ref,
               lnag_ref, lnab_ref, pe1w4_ref, pe1b_ref, pe2w_ref, pe2b_ref,
               xin_ref, xpe_ref):
    i = pl.program_id(0)
    nf = nf_ref[...]
    pnc = pnc_ref[...]
    feats = feats_ref[...]
    pself = pself_ref[...]

    a1 = jnp.maximum(pnc - pself[:, None, :] + lin1b_ref[...], 0.0)
    a1_2 = a1.reshape(BN * K, D)
    fc = jnp.dot(a1_2, lin2w_ref[...],
                 preferred_element_type=jnp.float32) + lin2b_ref[...]
    pe = fc + nf.reshape(BN * K, D)
    q = jnp.maximum(jnp.dot(feats, qw_ref[...],
                            preferred_element_type=jnp.float32)
                    + qb_ref[...], 0.0)
    kk = jnp.maximum(jnp.dot(pe, kw_ref[...],
                             preferred_element_type=jnp.float32)
                     + kb_ref[...], 0.0)
    vv = jnp.dot(pe, vw_ref[...],
                 preferred_element_type=jnp.float32) + vb_ref[...]
    qrep = jnp.broadcast_to(q[:, None, :], (BN, K, D)).reshape(BN * K, D)
    rel = (kk - qrep) * fc
    w1 = jnp.maximum(jnp.dot(rel, we1w_ref[...],
                             preferred_element_type=jnp.float32)
                     + we1b_ref[...], 0.0)
    z = (jnp.dot(w1, we2w_ref[...], preferred_element_type=jnp.float32)
         + we2b_ref[...]).reshape(BN, K, D)

    zmax = jnp.max(z, axis=1, keepdims=True)
    lo0 = zmax - 1.0

    def bis(_, lohi):
        lo, hi = lohi
        mid = 0.5 * (lo + hi)
        s = jnp.sum(jnp.maximum(z - mid, 0.0), axis=1, keepdims=True)
        go_up = s >= 1.0
        return jnp.where(go_up, mid, lo), jnp.where(go_up, hi, mid)

    lo, hi = lax.fori_loop(0, 18, bis, (lo0, zmax))
    tau = lo
    for _ in range(2):
        sup = (z > tau).astype(jnp.float32)
        cnt = jnp.sum(sup, axis=1, keepdims=True)
        ssum = jnp.sum(z * sup, axis=1, keepdims=True)
        tau = (ssum - 1.0) / cnt
    wgt = jnp.maximum(z - tau, 0.0)

    attn = jnp.sum(vv.reshape(BN, K, D) * wgt, axis=1)
    x_att = feats + attn
    xin = _ln(x_att, lnag_ref[...], lnab_ref[...])
    c4 = coords4_ref[pl.ds(i * BN, BN), :]
    cf1 = jnp.maximum(jnp.dot(c4, pe1w4_ref[...],
                              preferred_element_type=jnp.float32)
                      + pe1b_ref[...], 0.0)
    cf = jnp.dot(cf1, pe2w_ref[...],
                 preferred_element_type=jnp.float32) + pe2b_ref[...]
    xin_ref[...] = xin
    xpe_ref[...] = xin + cf


def _attn(nf, pnc, feats_p, pself, coords4, lin1_b, lin2_w, lin2_b,
          q_w, q_b, k_w, k_b, v_w, v_b, we1_w, we1_b, we2_w, we2_b,
          ln_att_g, ln_att_b, pe1_w4, pe1_b, pe2_w, pe2_b):
    full = pl.BlockSpec(memory_space=pltpu.VMEM)
    row1 = lambda arr: pl.BlockSpec((1, D), lambda i: (0, 0))
    in_specs = [
        pl.BlockSpec((BN, K, D), lambda i: (i, 0, 0)),
        pl.BlockSpec((BN, K, D), lambda i: (i, 0, 0)),
        pl.BlockSpec((BN, D), lambda i: (i, 0)),
        pl.BlockSpec((BN, D), lambda i: (i, 0)),
        full,
    ] + [full] * 19
    return pl.pallas_call(
        _attn_body,
        grid=(NP // BN,),
        in_specs=in_specs,
        out_specs=[
            pl.BlockSpec((BN, D), lambda i: (i, 0)),
            pl.BlockSpec((BN, D), lambda i: (i, 0)),
        ],
        out_shape=[
            jax.ShapeDtypeStruct((NP, D), jnp.float32),
            jax.ShapeDtypeStruct((NP, D), jnp.float32),
        ],
    )(nf.reshape(NP, K, D), pnc.reshape(NP, K, D), feats_p, pself, coords4,
      lin1_b.reshape(1, D), lin2_w, lin2_b.reshape(1, D),
      q_w, q_b.reshape(1, D), k_w, k_b.reshape(1, D), v_w, v_b.reshape(1, D),
      we1_w, we1_b.reshape(1, D), we2_w, we2_b.reshape(1, D),
      ln_att_g.reshape(1, D), ln_att_b.reshape(1, D),
      pe1_w4, pe1_b.reshape(1, D), pe2_w, pe2_b.reshape(1, D))


def _aggr_body(tbl_hbm, idxt_hbm, out_hbm, idxv, acc, sem):
    wid = lax.axis_index("s") * NC + lax.axis_index("c")
    base = wid * NPW
    pltpu.sync_copy(tbl_hbm.at[pl.ds(base, NPW)], acc)

    @pl.loop(0, K)
    def _nbr(j):
        pltpu.sync_copy(idxt_hbm.at[pl.ds(j * NP + base, NPW)], idxv)
        pltpu.async_copy(tbl_hbm.at[idxv], acc, sem, add=True).wait()

    pltpu.sync_copy(acc, out_hbm.at[pl.ds(base, NPW)])


def _aggregate(tbl, idx_t):
    mesh = plsc.VectorSubcoreMesh(core_axis_name="c", subcore_axis_name="s")
    fn = pl.kernel(
        _aggr_body,
        out_type=jax.ShapeDtypeStruct((NP, D), jnp.float32),
        mesh=mesh,
        scratch_types=[
            pltpu.VMEM((NPW,), jnp.int32),
            pltpu.VMEM((NPW, D), jnp.float32),
            pltpu.SemaphoreType.DMA,
        ],
    )
    return fn(tbl, idx_t)


def _lin_relu_body(x_ref, w_ref, b_ref, o_ref):
    o_ref[...] = jnp.maximum(
        jnp.dot(x_ref[...], w_ref[...], preferred_element_type=jnp.float32)
        + b_ref[...], 0.0)


def _lin_relu(x, w, b):
    return pl.pallas_call(
        _lin_relu_body,
        grid=(NP // RB,),
        in_specs=[
            pl.BlockSpec((RB, D), lambda i: (i, 0)),
            pl.BlockSpec(memory_space=pltpu.VMEM),
            pl.BlockSpec(memory_space=pltpu.VMEM),
        ],
        out_specs=pl.BlockSpec((RB, D), lambda i: (i, 0)),
        out_shape=jax.ShapeDtypeStruct((NP, D), jnp.float32),
    )(x, w, b.reshape(1, D))


def _lin_res_ln_body(x_ref, w_ref, b_ref, res_ref, g_ref, bb_ref, o_ref):
    h = (jnp.dot(x_ref[...], w_ref[...], preferred_element_type=jnp.float32)
         + b_ref[...]) + res_ref[...]
    o_ref[...] = _ln(h, g_ref[...], bb_ref[...])


def _lin_res_ln(x, w, b, res, g, bb):
    return pl.pallas_call(
        _lin_res_ln_body,
        grid=(NP // RB,),
        in_specs=[
            pl.BlockSpec((RB, D), lambda i: (i, 0)),
            pl.BlockSpec(memory_space=pltpu.VMEM),
            pl.BlockSpec(memory_space=pltpu.VMEM),
            pl.BlockSpec((RB, D), lambda i: (i, 0)),
            pl.BlockSpec(memory_space=pltpu.VMEM),
            pl.BlockSpec(memory_space=pltpu.VMEM),
        ],
        out_specs=pl.BlockSpec((RB, D), lambda i: (i, 0)),
        out_shape=jax.ShapeDtypeStruct((NP, D), jnp.float32),
    )(x, w, b.reshape(1, D), res, g.reshape(1, D), bb.reshape(1, D))


def kernel(coords, feats, q_w, q_b, k_w, k_b, v_w, v_b, lin1_w, lin1_b,
           lin2_w, lin2_b, we1_w, we1_b, we2_w, we2_b, ln_att_g, ln_att_b,
           pe1_w, pe1_b, pe2_w, pe2_b, sg1_w, sg1_b, sg2_w, sg2_b,
           ln_ff_g, ln_ff_b):
    f32 = jnp.float32
    pad = NP - N
    coords4 = jnp.concatenate(
        [jnp.concatenate([coords, jnp.full((pad, 3), 1e4, f32)], axis=0),
         jnp.zeros((NP, 1), f32)], axis=1)
    feats_p = jnp.concatenate([feats, jnp.zeros((pad, D), f32)], axis=0)
    lin1_w4 = jnp.concatenate([lin1_w, jnp.zeros((1, D), f32)], axis=0)
    pe1_w4 = jnp.concatenate([pe1_w, jnp.zeros((1, D), f32)], axis=0)

    idx2, p_tbl = _knn(coords4, lin1_w4)
    idx_flat = idx2[:, 0:K].reshape(NP * K)
    idx_gt = idx2[:, K:2 * K].T.reshape(K * NP)

    nf, pnc = _gather2(feats_p, p_tbl, idx_flat)

    xin, xpe = _attn(nf, pnc, feats_p, p_tbl, coords4,
                     lin1_b, lin2_w, lin2_b, q_w, q_b, k_w, k_b, v_w, v_b,
                     we1_w, we1_b, we2_w, we2_b, ln_att_g, ln_att_b,
                     pe1_w4, pe1_b, pe2_w, pe2_b)

    agg1 = _aggregate(xpe, idx_gt)
    h = _lin_relu(agg1, sg1_w * (1.0 / 33.0), sg1_b)
    agg2 = _aggregate(h, idx_gt)
    out = _lin_res_ln(agg2, sg2_w * (1.0 / 33.0), sg2_b, xin,
                      ln_ff_g, ln_ff_b)
    return out[:N]


# speedup vs baseline: 8.1542x; 1.2254x over previous
import functools

import jax
import jax.numpy as jnp
from jax import lax
from jax.experimental import pallas as pl
from jax.experimental.pallas import tpu as pltpu
from jax.experimental.pallas import tpu_sc as plsc

N = 10000
NP = 10240
D = 128
K = 32
T = 33
QB = 128
BN = 64
RB = 1024
NC, NS = 2, 16
NW = NC * NS
NPW = NP // NW
GCH = 16

_BIGF = 1e9
_INF = 3e38


def _knn_body(coords4_ref, ct_ref, lin1w4_ref, idx_ref, p_ref, d_scr):
    i = pl.program_id(0)
    q = coords4_ref[pl.ds(i * QB, QB), :]
    ct = ct_ref[...]
    qc = jnp.dot(q, ct, preferred_element_type=jnp.float32)
    qsq = jnp.sum(q * q, axis=1, keepdims=True)
    csq = jnp.sum(ct * ct, axis=0, keepdims=True)
    d_scr[...] = qsq - 2.0 * qc + csq

    col64 = lax.broadcasted_iota(jnp.int32, (QB, 64), 1)

    def step(t, e):
        d = d_scr[...]
        m = jnp.min(d, axis=1, keepdims=True)
        iota_j = lax.broadcasted_iota(jnp.int32, (QB, NP), 1)
        masked = jnp.where(d == m, iota_j, jnp.int32(2**30))
        jsel = jnp.min(masked, axis=1, keepdims=True)
        d_scr[...] = jnp.where(masked == jsel, _INF, d)
        return jnp.where(col64 == t, jsel, e)

    e = lax.fori_loop(0, T, step, jnp.zeros((QB, 64), jnp.int32))

    rowid = i * QB + lax.broadcasted_iota(jnp.int32, (QB, 1), 0)
    self_hit = (e == rowid) & (col64 < T)
    p = jnp.min(jnp.where(self_hit, col64, jnp.int32(10 * T)), axis=1,
                keepdims=True)
    idx = e[:, 0:K]
    idx_g = jnp.where(col64[:, 0:K] < p, idx, e[:, 1:K + 1])
    idx_ref[:, 0:K] = idx
    idx_ref[:, K:2 * K] = idx_g
    p_ref[...] = jnp.dot(q, lin1w4_ref[...],
                         preferred_element_type=jnp.float32)


def _knn(coords4, lin1_w4):
    ct = coords4[:, 0:4].T
    return pl.pallas_call(
        _knn_body,
        grid=(NP // QB,),
        in_specs=[
            pl.BlockSpec(memory_space=pltpu.VMEM),
            pl.BlockSpec(memory_space=pltpu.VMEM),
            pl.BlockSpec(memory_space=pltpu.VMEM),
        ],
        out_specs=[
            pl.BlockSpec((QB, 128), lambda i: (i, 0)),
            pl.BlockSpec((QB, D), lambda i: (i, 0)),
        ],
        out_shape=[
            jax.ShapeDtypeStruct((NP, 128), jnp.int32),
            jax.ShapeDtypeStruct((NP, D), jnp.float32),
        ],
        scratch_shapes=[pltpu.VMEM((QB, NP), jnp.float32)],
    )(coords4, ct, lin1_w4)


def _gather2_body(t1_hbm, t2_hbm, idx_hbm, o1_hbm, o2_hbm, idxv, buf, sem):
    wid = lax.axis_index("s") * NC + lax.axis_index("c")
    rows_per_w = NPW * K
    base = wid * rows_per_w
    ch = GCH * K

    @pl.loop(0, rows_per_w, step=ch)
    def _chunk(off):
        start = base + off
        pltpu.sync_copy(idx_hbm.at[pl.ds(start, ch)], idxv)
        pltpu.async_copy(t1_hbm.at[idxv], buf, sem).wait()
        pltpu.sync_copy(buf, o1_hbm.at[pl.ds(start, ch)])
        pltpu.async_copy(t2_hbm.at[idxv], buf, sem).wait()
        pltpu.sync_copy(buf, o2_hbm.at[pl.ds(start, ch)])


def _gather2(tbl1, tbl2, idx_flat):
    mesh = plsc.VectorSubcoreMesh(core_axis_name="c", subcore_axis_name="s")
    fn = pl.kernel(
        _gather2_body,
        out_type=[
            jax.ShapeDtypeStruct((NP * K, D), jnp.float32),
            jax.ShapeDtypeStruct((NP * K, D), jnp.float32),
        ],
        mesh=mesh,
        scratch_types=[
            pltpu.VMEM((GCH * K,), jnp.int32),
            pltpu.VMEM((GCH * K, D), jnp.float32),
            pltpu.SemaphoreType.DMA,
        ],
    )
    return fn(tbl1, tbl2, idx_flat)


def _ln(x, g, b):
    m = jnp.mean(x, -1, keepdims=True)
    v = jnp.mean((x - m) ** 2, -1, keepdims=True)
    return (x - m) / jnp.sqrt(v + 1e-5) * g + b


def _attn_body(nf_ref, pnc_ref, feats_ref, pself_ref, coords4_ref,
               lin1b_ref, lin2w_ref, lin2b_ref,
               qw_ref, qb_ref, kw_ref, kb_ref, vw_ref, vb_ref,
               we1w_ref, we1b_ref, we2w_ref, we2b_ref,
               lnag_ref, lnab_ref, pe1w4_ref, pe1b_ref, pe2w_ref, pe2b_ref,
               xin_ref, xpe_ref):
    i = pl.program_id(0)
    nf = nf_ref[...]
    pnc = pnc_ref[...]
    feats = feats_ref[...]
    pself = pself_ref[...]

    a1 = jnp.maximum(pnc - pself[:, None, :] + lin1b_ref[...], 0.0)
    a1_2 = a1.reshape(BN * K, D)
    fc = jnp.dot(a1_2, lin2w_ref[...],
                 preferred_element_type=jnp.float32) + lin2b_ref[...]
    pe = fc + nf.reshape(BN * K, D)
    q = jnp.maximum(jnp.dot(feats, qw_ref[...],
                            preferred_element_type=jnp.float32)
                    + qb_ref[...], 0.0)
    kk = jnp.maximum(jnp.dot(pe, kw_ref[...],
                             preferred_element_type=jnp.float32)
                     + kb_ref[...], 0.0)
    vv = jnp.dot(pe, vw_ref[...],
                 preferred_element_type=jnp.float32) + vb_ref[...]
    qrep = jnp.broadcast_to(q[:, None, :], (BN, K, D)).reshape(BN * K, D)
    rel = (kk - qrep) * fc
    w1 = jnp.maximum(jnp.dot(rel, we1w_ref[...],
                             preferred_element_type=jnp.float32)
                     + we1b_ref[...], 0.0)
    z = (jnp.dot(w1, we2w_ref[...], preferred_element_type=jnp.float32)
         + we2b_ref[...]).reshape(BN, K, D)

    zmax = jnp.max(z, axis=1, keepdims=True)
    lo0 = zmax - 1.0

    def bis(_, lohi):
        lo, hi = lohi
        mid = 0.5 * (lo + hi)
        s = jnp.sum(jnp.maximum(z - mid, 0.0), axis=1, keepdims=True)
        go_up = s >= 1.0
        return jnp.where(go_up, mid, lo), jnp.where(go_up, hi, mid)

    lo, hi = lax.fori_loop(0, 14, bis, (lo0, zmax))
    tau = lo
    for _ in range(2):
        sup = (z > tau).astype(jnp.float32)
        cnt = jnp.sum(sup, axis=1, keepdims=True)
        ssum = jnp.sum(z * sup, axis=1, keepdims=True)
        tau = (ssum - 1.0) / cnt
    wgt = jnp.maximum(z - tau, 0.0)

    attn = jnp.sum(vv.reshape(BN, K, D) * wgt, axis=1)
    x_att = feats + attn
    xin = _ln(x_att, lnag_ref[...], lnab_ref[...])
    c4 = coords4_ref[pl.ds(i * BN, BN), :]
    cf1 = jnp.maximum(jnp.dot(c4, pe1w4_ref[...],
                              preferred_element_type=jnp.float32)
                      + pe1b_ref[...], 0.0)
    cf = jnp.dot(cf1, pe2w_ref[...],
                 preferred_element_type=jnp.float32) + pe2b_ref[...]
    xin_ref[...] = xin
    xpe_ref[...] = xin + cf


def _attn(nf, pnc, feats_p, pself, coords4, lin1_b, lin2_w, lin2_b,
          q_w, q_b, k_w, k_b, v_w, v_b, we1_w, we1_b, we2_w, we2_b,
          ln_att_g, ln_att_b, pe1_w4, pe1_b, pe2_w, pe2_b):
    full = pl.BlockSpec(memory_space=pltpu.VMEM)
    row1 = lambda arr: pl.BlockSpec((1, D), lambda i: (0, 0))
    in_specs = [
        pl.BlockSpec((BN, K, D), lambda i: (i, 0, 0)),
        pl.BlockSpec((BN, K, D), lambda i: (i, 0, 0)),
        pl.BlockSpec((BN, D), lambda i: (i, 0)),
        pl.BlockSpec((BN, D), lambda i: (i, 0)),
        full,
    ] + [full] * 19
    return pl.pallas_call(
        _attn_body,
        grid=(NP // BN,),
        in_specs=in_specs,
        out_specs=[
            pl.BlockSpec((BN, D), lambda i: (i, 0)),
            pl.BlockSpec((BN, D), lambda i: (i, 0)),
        ],
        out_shape=[
            jax.ShapeDtypeStruct((NP, D), jnp.float32),
            jax.ShapeDtypeStruct((NP, D), jnp.float32),
        ],
    )(nf.reshape(NP, K, D), pnc.reshape(NP, K, D), feats_p, pself, coords4,
      lin1_b.reshape(1, D), lin2_w, lin2_b.reshape(1, D),
      q_w, q_b.reshape(1, D), k_w, k_b.reshape(1, D), v_w, v_b.reshape(1, D),
      we1_w, we1_b.reshape(1, D), we2_w, we2_b.reshape(1, D),
      ln_att_g.reshape(1, D), ln_att_b.reshape(1, D),
      pe1_w4, pe1_b.reshape(1, D), pe2_w, pe2_b.reshape(1, D))


def _aggr_body(tbl_hbm, idxt_hbm, out_hbm, idxv, acc, sem):
    wid = lax.axis_index("s") * NC + lax.axis_index("c")
    base = wid * NPW
    pltpu.sync_copy(tbl_hbm.at[pl.ds(base, NPW)], acc)

    @pl.loop(0, K)
    def _nbr(j):
        pltpu.sync_copy(idxt_hbm.at[pl.ds(j * NP + base, NPW)], idxv)
        pltpu.async_copy(tbl_hbm.at[idxv], acc, sem, add=True).wait()

    pltpu.sync_copy(acc, out_hbm.at[pl.ds(base, NPW)])


def _aggregate(tbl, idx_t):
    mesh = plsc.VectorSubcoreMesh(core_axis_name="c", subcore_axis_name="s")
    fn = pl.kernel(
        _aggr_body,
        out_type=jax.ShapeDtypeStruct((NP, D), jnp.float32),
        mesh=mesh,
        scratch_types=[
            pltpu.VMEM((NPW,), jnp.int32),
            pltpu.VMEM((NPW, D), jnp.float32),
            pltpu.SemaphoreType.DMA,
        ],
    )
    return fn(tbl, idx_t)


def _lin_relu_body(x_ref, w_ref, b_ref, o_ref):
    o_ref[...] = jnp.maximum(
        jnp.dot(x_ref[...], w_ref[...], preferred_element_type=jnp.float32)
        + b_ref[...], 0.0)


def _lin_relu(x, w, b):
    return pl.pallas_call(
        _lin_relu_body,
        grid=(NP // RB,),
        in_specs=[
            pl.BlockSpec((RB, D), lambda i: (i, 0)),
            pl.BlockSpec(memory_space=pltpu.VMEM),
            pl.BlockSpec(memory_space=pltpu.VMEM),
        ],
        out_specs=pl.BlockSpec((RB, D), lambda i: (i, 0)),
        out_shape=jax.ShapeDtypeStruct((NP, D), jnp.float32),
    )(x, w, b.reshape(1, D))


def _lin_res_ln_body(x_ref, w_ref, b_ref, res_ref, g_ref, bb_ref, o_ref):
    h = (jnp.dot(x_ref[...], w_ref[...], preferred_element_type=jnp.float32)
         + b_ref[...]) + res_ref[...]
    o_ref[...] = _ln(h, g_ref[...], bb_ref[...])


def _lin_res_ln(x, w, b, res, g, bb):
    return pl.pallas_call(
        _lin_res_ln_body,
        grid=(NP // RB,),
        in_specs=[
            pl.BlockSpec((RB, D), lambda i: (i, 0)),
            pl.BlockSpec(memory_space=pltpu.VMEM),
            pl.BlockSpec(memory_space=pltpu.VMEM),
            pl.BlockSpec((RB, D), lambda i: (i, 0)),
            pl.BlockSpec(memory_space=pltpu.VMEM),
            pl.BlockSpec(memory_space=pltpu.VMEM),
        ],
        out_specs=pl.BlockSpec((RB, D), lambda i: (i, 0)),
        out_shape=jax.ShapeDtypeStruct((NP, D), jnp.float32),
    )(x, w, b.reshape(1, D), res, g.reshape(1, D), bb.reshape(1, D))


def kernel(coords, feats, q_w, q_b, k_w, k_b, v_w, v_b, lin1_w, lin1_b,
           lin2_w, lin2_b, we1_w, we1_b, we2_w, we2_b, ln_att_g, ln_att_b,
           pe1_w, pe1_b, pe2_w, pe2_b, sg1_w, sg1_b, sg2_w, sg2_b,
           ln_ff_g, ln_ff_b):
    f32 = jnp.float32
    pad = NP - N
    coords4 = jnp.concatenate(
        [jnp.concatenate([coords, jnp.full((pad, 3), 1e4, f32)], axis=0),
         jnp.zeros((NP, 1), f32)], axis=1)
    feats_p = jnp.concatenate([feats, jnp.zeros((pad, D), f32)], axis=0)
    lin1_w4 = jnp.concatenate([lin1_w, jnp.zeros((1, D), f32)], axis=0)
    pe1_w4 = jnp.concatenate([pe1_w, jnp.zeros((1, D), f32)], axis=0)

    idx2, p_tbl = _knn(coords4, lin1_w4)
    idx_flat = idx2[:, 0:K].reshape(NP * K)
    idx_gt = idx2[:, K:2 * K].T.reshape(K * NP)

    nf, pnc = _gather2(feats_p, p_tbl, idx_flat)

    xin, xpe = _attn(nf, pnc, feats_p, p_tbl, coords4,
                     lin1_b, lin2_w, lin2_b, q_w, q_b, k_w, k_b, v_w, v_b,
                     we1_w, we1_b, we2_w, we2_b, ln_att_g, ln_att_b,
                     pe1_w4, pe1_b, pe2_w, pe2_b)

    agg1 = _aggregate(xpe, idx_gt)
    h = _lin_relu(agg1, sg1_w * (1.0 / 33.0), sg1_b)
    agg2 = _aggregate(h, idx_gt)
    out = _lin_res_ln(agg2, sg2_w * (1.0 / 33.0), sg2_b, xin,
                      ln_ff_g, ln_ff_b)
    return out[:N]


# speedup vs baseline: 8.5415x; 1.0475x over previous
import functools

import jax
import jax.numpy as jnp
from jax import lax
from jax.experimental import pallas as pl
from jax.experimental.pallas import tpu as pltpu
from jax.experimental.pallas import tpu_sc as plsc

N = 10000
NP = 10240
D = 128
K = 32
T = 33
QB = 128
BN = 64
RB = 1024
NC, NS = 2, 16
NW = NC * NS
NPW = NP // NW
GCH = 10

_BIGF = 1e9
_INF = 3e38


def _knn_body(coords4_ref, ct_ref, lin1w4_ref, idx_ref, p_ref, d_scr):
    i = pl.program_id(0)
    q = coords4_ref[pl.ds(i * QB, QB), :]
    ct = ct_ref[...]
    qc = jnp.dot(q, ct, preferred_element_type=jnp.float32)
    qsq = jnp.sum(q * q, axis=1, keepdims=True)
    csq = jnp.sum(ct * ct, axis=0, keepdims=True)
    d_scr[...] = qsq - 2.0 * qc + csq

    col64 = lax.broadcasted_iota(jnp.int32, (QB, 64), 1)

    def step(t, e):
        d = d_scr[...]
        m = jnp.min(d, axis=1, keepdims=True)
        iota_j = lax.broadcasted_iota(jnp.int32, (QB, NP), 1)
        masked = jnp.where(d == m, iota_j, jnp.int32(2**30))
        jsel = jnp.min(masked, axis=1, keepdims=True)
        d_scr[...] = jnp.where(masked == jsel, _INF, d)
        return jnp.where(col64 == t, jsel, e)

    e = lax.fori_loop(0, T, step, jnp.zeros((QB, 64), jnp.int32))

    rowid = i * QB + lax.broadcasted_iota(jnp.int32, (QB, 1), 0)
    self_hit = (e == rowid) & (col64 < T)
    p = jnp.min(jnp.where(self_hit, col64, jnp.int32(10 * T)), axis=1,
                keepdims=True)
    idx = e[:, 0:K]
    idx_g = jnp.where(col64[:, 0:K] < p, idx, e[:, 1:K + 1])
    idx_ref[:, 0:K] = idx
    idx_ref[:, K:2 * K] = idx_g
    p_ref[...] = jnp.dot(q, lin1w4_ref[...],
                         preferred_element_type=jnp.float32)


def _knn(coords4, lin1_w4):
    ct = coords4[:, 0:4].T
    return pl.pallas_call(
        _knn_body,
        grid=(NP // QB,),
        in_specs=[
            pl.BlockSpec(memory_space=pltpu.VMEM),
            pl.BlockSpec(memory_space=pltpu.VMEM),
            pl.BlockSpec(memory_space=pltpu.VMEM),
        ],
        out_specs=[
            pl.BlockSpec((QB, 128), lambda i: (i, 0)),
            pl.BlockSpec((QB, D), lambda i: (i, 0)),
        ],
        out_shape=[
            jax.ShapeDtypeStruct((NP, 128), jnp.int32),
            jax.ShapeDtypeStruct((NP, D), jnp.float32),
        ],
        scratch_shapes=[pltpu.VMEM((QB, NP), jnp.float32)],
    )(coords4, ct, lin1_w4)


def _gather2_body(t1_hbm, t2_hbm, idx_hbm, o1_hbm, o2_hbm, idxv, bufa, bufb,
                  sema, semb):
    wid = lax.axis_index("s") * NC + lax.axis_index("c")
    rows_per_w = NPW * K
    base = wid * rows_per_w
    ch = GCH * K

    @pl.loop(0, rows_per_w, step=ch)
    def _chunk(off):
        start = base + off
        pltpu.sync_copy(idx_hbm.at[pl.ds(start, ch)], idxv)
        ca = pltpu.async_copy(t1_hbm.at[idxv], bufa, sema)
        cb = pltpu.async_copy(t2_hbm.at[idxv], bufb, semb)
        ca.wait()
        pltpu.sync_copy(bufa, o1_hbm.at[pl.ds(start, ch)])
        cb.wait()
        pltpu.sync_copy(bufb, o2_hbm.at[pl.ds(start, ch)])


def _gather2(tbl1, tbl2, idx_flat):
    mesh = plsc.VectorSubcoreMesh(core_axis_name="c", subcore_axis_name="s")
    fn = pl.kernel(
        _gather2_body,
        out_type=[
            jax.ShapeDtypeStruct((NP * K, D), jnp.float32),
            jax.ShapeDtypeStruct((NP * K, D), jnp.float32),
        ],
        mesh=mesh,
        scratch_types=[
            pltpu.VMEM((GCH * K,), jnp.int32),
            pltpu.VMEM((GCH * K, D), jnp.float32),
            pltpu.VMEM((GCH * K, D), jnp.float32),
            pltpu.SemaphoreType.DMA,
            pltpu.SemaphoreType.DMA,
        ],
    )
    return fn(tbl1, tbl2, idx_flat)


def _ln(x, g, b):
    m = jnp.mean(x, -1, keepdims=True)
    v = jnp.mean((x - m) ** 2, -1, keepdims=True)
    return (x - m) / jnp.sqrt(v + 1e-5) * g + b


def _attn_body(nf_ref, pnc_ref, feats_ref, pself_ref, coords4_ref,
               lin1b_ref, lin2w_ref, lin2b_ref,
               qw_ref, qb_ref, kw_ref, kb_ref, vw_ref, vb_ref,
               we1w_ref, we1b_ref, we2w_ref, we2b_ref,
               lnag_ref, lnab_ref, pe1w4_ref, pe1b_ref, pe2w_ref, pe2b_ref,
               xin_ref, xpe_ref):
    i = pl.program_id(0)
    nf = nf_ref[...]
    pnc = pnc_ref[...]
    feats = feats_ref[...]
    pself = pself_ref[...]

    a1 = jnp.maximum(pnc - pself[:, None, :] + lin1b_ref[...], 0.0)
    a1_2 = a1.reshape(BN * K, D)
    fc = jnp.dot(a1_2, lin2w_ref[...],
                 preferred_element_type=jnp.float32) + lin2b_ref[...]
    pe = fc + nf.reshape(BN * K, D)
    q = jnp.maximum(jnp.dot(feats, qw_ref[...],
                            preferred_element_type=jnp.float32)
                    + qb_ref[...], 0.0)
    kk = jnp.maximum(jnp.dot(pe, kw_ref[...],
                             preferred_element_type=jnp.float32)
                     + kb_ref[...], 0.0)
    vv = jnp.dot(pe, vw_ref[...],
                 preferred_element_type=jnp.float32) + vb_ref[...]
    qrep = jnp.broadcast_to(q[:, None, :], (BN, K, D)).reshape(BN * K, D)
    rel = (kk - qrep) * fc
    w1 = jnp.maximum(jnp.dot(rel, we1w_ref[...],
                             preferred_element_type=jnp.float32)
                     + we1b_ref[...], 0.0)
    z = (jnp.dot(w1, we2w_ref[...], preferred_element_type=jnp.float32)
         + we2b_ref[...]).reshape(BN, K, D)

    zmax = jnp.max(z, axis=1, keepdims=True)
    lo0 = zmax - 1.0

    def bis(_, lohi):
        lo, hi = lohi
        mid = 0.5 * (lo + hi)
        s = jnp.sum(jnp.maximum(z - mid, 0.0), axis=1, keepdims=True)
        go_up = s >= 1.0
        return jnp.where(go_up, mid, lo), jnp.where(go_up, hi, mid)

    lo, hi = lax.fori_loop(0, 14, bis, (lo0, zmax))
    tau = lo
    for _ in range(2):
        sup = (z > tau).astype(jnp.float32)
        cnt = jnp.sum(sup, axis=1, keepdims=True)
        ssum = jnp.sum(z * sup, axis=1, keepdims=True)
        tau = (ssum - 1.0) / cnt
    wgt = jnp.maximum(z - tau, 0.0)

    attn = jnp.sum(vv.reshape(BN, K, D) * wgt, axis=1)
    x_att = feats + attn
    xin = _ln(x_att, lnag_ref[...], lnab_ref[...])
    c4 = coords4_ref[pl.ds(i * BN, BN), :]
    cf1 = jnp.maximum(jnp.dot(c4, pe1w4_ref[...],
                              preferred_element_type=jnp.float32)
                      + pe1b_ref[...], 0.0)
    cf = jnp.dot(cf1, pe2w_ref[...],
                 preferred_element_type=jnp.float32) + pe2b_ref[...]
    xin_ref[...] = xin
    xpe_ref[...] = xin + cf


def _attn(nf, pnc, feats_p, pself, coords4, lin1_b, lin2_w, lin2_b,
          q_w, q_b, k_w, k_b, v_w, v_b, we1_w, we1_b, we2_w, we2_b,
          ln_att_g, ln_att_b, pe1_w4, pe1_b, pe2_w, pe2_b):
    full = pl.BlockSpec(memory_space=pltpu.VMEM)
    row1 = lambda arr: pl.BlockSpec((1, D), lambda i: (0, 0))
    in_specs = [
        pl.BlockSpec((BN, K, D), lambda i: (i, 0, 0)),
        pl.BlockSpec((BN, K, D), lambda i: (i, 0, 0)),
        pl.BlockSpec((BN, D), lambda i: (i, 0)),
        pl.BlockSpec((BN, D), lambda i: (i, 0)),
        full,
    ] + [full] * 19
    return pl.pallas_call(
        _attn_body,
        grid=(NP // BN,),
        in_specs=in_specs,
        out_specs=[
            pl.BlockSpec((BN, D), lambda i: (i, 0)),
            pl.BlockSpec((BN, D), lambda i: (i, 0)),
        ],
        out_shape=[
            jax.ShapeDtypeStruct((NP, D), jnp.float32),
            jax.ShapeDtypeStruct((NP, D), jnp.float32),
        ],
    )(nf.reshape(NP, K, D), pnc.reshape(NP, K, D), feats_p, pself, coords4,
      lin1_b.reshape(1, D), lin2_w, lin2_b.reshape(1, D),
      q_w, q_b.reshape(1, D), k_w, k_b.reshape(1, D), v_w, v_b.reshape(1, D),
      we1_w, we1_b.reshape(1, D), we2_w, we2_b.reshape(1, D),
      ln_att_g.reshape(1, D), ln_att_b.reshape(1, D),
      pe1_w4, pe1_b.reshape(1, D), pe2_w, pe2_b.reshape(1, D))


def _aggr_body(tbl_hbm, idxt_hbm, oa_hbm, ob_hbm, idxva, idxvb, acca, accb,
               sema, semb):
    wid = lax.axis_index("s") * NC + lax.axis_index("c")
    base = wid * NPW
    pltpu.sync_copy(tbl_hbm.at[pl.ds(base, NPW)], acca)
    pltpu.sync_copy(idxt_hbm.at[pl.ds(base, NPW)], idxvb)
    pltpu.async_copy(tbl_hbm.at[idxvb], accb, semb).wait()

    @pl.loop(1, K - 1, step=2)
    def _nbr(j):
        pltpu.sync_copy(idxt_hbm.at[pl.ds(j * NP + base, NPW)], idxva)
        ca = pltpu.async_copy(tbl_hbm.at[idxva], acca, sema, add=True)
        pltpu.sync_copy(idxt_hbm.at[pl.ds((j + 1) * NP + base, NPW)], idxvb)
        cb = pltpu.async_copy(tbl_hbm.at[idxvb], accb, semb, add=True)
        ca.wait()
        cb.wait()

    pltpu.sync_copy(idxt_hbm.at[pl.ds((K - 1) * NP + base, NPW)], idxva)
    pltpu.async_copy(tbl_hbm.at[idxva], acca, sema, add=True).wait()
    pltpu.sync_copy(acca, oa_hbm.at[pl.ds(base, NPW)])
    pltpu.sync_copy(accb, ob_hbm.at[pl.ds(base, NPW)])


def _aggregate(tbl, idx_t):
    mesh = plsc.VectorSubcoreMesh(core_axis_name="c", subcore_axis_name="s")
    fn = pl.kernel(
        _aggr_body,
        out_type=[
            jax.ShapeDtypeStruct((NP, D), jnp.float32),
            jax.ShapeDtypeStruct((NP, D), jnp.float32),
        ],
        mesh=mesh,
        scratch_types=[
            pltpu.VMEM((NPW,), jnp.int32),
            pltpu.VMEM((NPW,), jnp.int32),
            pltpu.VMEM((NPW, D), jnp.float32),
            pltpu.VMEM((NPW, D), jnp.float32),
            pltpu.SemaphoreType.DMA,
            pltpu.SemaphoreType.DMA,
        ],
    )
    return fn(tbl, idx_t)


def _lin_relu_body(xa_ref, xb_ref, w_ref, b_ref, o_ref):
    o_ref[...] = jnp.maximum(
        jnp.dot(xa_ref[...] + xb_ref[...], w_ref[...],
                preferred_element_type=jnp.float32) + b_ref[...], 0.0)


def _lin_relu(xa, xb, w, b):
    return pl.pallas_call(
        _lin_relu_body,
        grid=(NP // RB,),
        in_specs=[
            pl.BlockSpec((RB, D), lambda i: (i, 0)),
            pl.BlockSpec((RB, D), lambda i: (i, 0)),
            pl.BlockSpec(memory_space=pltpu.VMEM),
            pl.BlockSpec(memory_space=pltpu.VMEM),
        ],
        out_specs=pl.BlockSpec((RB, D), lambda i: (i, 0)),
        out_shape=jax.ShapeDtypeStruct((NP, D), jnp.float32),
    )(xa, xb, w, b.reshape(1, D))


def _lin_res_ln_body(xa_ref, xb_ref, w_ref, b_ref, res_ref, g_ref, bb_ref,
                     o_ref):
    h = (jnp.dot(xa_ref[...] + xb_ref[...], w_ref[...],
                 preferred_element_type=jnp.float32)
         + b_ref[...]) + res_ref[...]
    o_ref[...] = _ln(h, g_ref[...], bb_ref[...])


def _lin_res_ln(xa, xb, w, b, res, g, bb):
    return pl.pallas_call(
        _lin_res_ln_body,
        grid=(NP // RB,),
        in_specs=[
            pl.BlockSpec((RB, D), lambda i: (i, 0)),
            pl.BlockSpec((RB, D), lambda i: (i, 0)),
            pl.BlockSpec(memory_space=pltpu.VMEM),
            pl.BlockSpec(memory_space=pltpu.VMEM),
            pl.BlockSpec((RB, D), lambda i: (i, 0)),
            pl.BlockSpec(memory_space=pltpu.VMEM),
            pl.BlockSpec(memory_space=pltpu.VMEM),
        ],
        out_specs=pl.BlockSpec((RB, D), lambda i: (i, 0)),
        out_shape=jax.ShapeDtypeStruct((NP, D), jnp.float32),
    )(xa, xb, w, b.reshape(1, D), res, g.reshape(1, D), bb.reshape(1, D))


def kernel(coords, feats, q_w, q_b, k_w, k_b, v_w, v_b, lin1_w, lin1_b,
           lin2_w, lin2_b, we1_w, we1_b, we2_w, we2_b, ln_att_g, ln_att_b,
           pe1_w, pe1_b, pe2_w, pe2_b, sg1_w, sg1_b, sg2_w, sg2_b,
           ln_ff_g, ln_ff_b):
    f32 = jnp.float32
    pad = NP - N
    coords4 = jnp.concatenate(
        [jnp.concatenate([coords, jnp.full((pad, 3), 1e4, f32)], axis=0),
         jnp.zeros((NP, 1), f32)], axis=1)
    feats_p = jnp.concatenate([feats, jnp.zeros((pad, D), f32)], axis=0)
    lin1_w4 = jnp.concatenate([lin1_w, jnp.zeros((1, D), f32)], axis=0)
    pe1_w4 = jnp.concatenate([pe1_w, jnp.zeros((1, D), f32)], axis=0)

    idx2, p_tbl = _knn(coords4, lin1_w4)
    idx_flat = idx2[:, 0:K].reshape(NP * K)
    idx_gt = idx2[:, K:2 * K].T.reshape(K * NP)

    nf, pnc = _gather2(feats_p, p_tbl, idx_flat)

    xin, xpe = _attn(nf, pnc, feats_p, p_tbl, coords4,
                     lin1_b, lin2_w, lin2_b, q_w, q_b, k_w, k_b, v_w, v_b,
                     we1_w, we1_b, we2_w, we2_b, ln_att_g, ln_att_b,
                     pe1_w4, pe1_b, pe2_w, pe2_b)

    agg1a, agg1b = _aggregate(xpe, idx_gt)
    h = _lin_relu(agg1a, agg1b, sg1_w * (1.0 / 33.0), sg1_b)
    agg2a, agg2b = _aggregate(h, idx_gt)
    out = _lin_res_ln(agg2a, agg2b, sg2_w * (1.0 / 33.0), sg2_b, xin,
                      ln_ff_g, ln_ff_b)
    return out[:N]
